# baseline (speedup 1.0000x reference)
import functools

import jax
import jax.numpy as jnp
from jax import lax
from jax.experimental import pallas as pl
from jax.experimental.pallas import tpu as pltpu
from jax.experimental.pallas import tpu_sc as plsc

_B, _S, _D = 4, 2048, 64
_NV = _S // 16


def _tc_stats(t_ref, obs_ref, sid_ref, vid_ref, n_ref, s1_ref, s2_ref,
              combo_ref, off_ref):
    t = t_ref[...]
    obs = obs_ref[...]
    tm = t * obs
    n_ref[...] = jnp.sum(obs, axis=-1)
    s1_ref[...] = jnp.sum(tm, axis=-1)
    s2_ref[...] = jnp.sum(tm * t, axis=-1)

    combo = sid_ref[...] * 8 + vid_ref[...]
    combo_ref[...] = combo
    iota64 = jax.lax.broadcasted_iota(jnp.int32, (1, 1, 64), 2)
    lt = (combo[:, :, None] < iota64).astype(jnp.float32)
    off_ref[...] = jnp.sum(lt, axis=1).astype(jnp.int32)


def _newton_sqrt(w):
    bits = plsc.bitcast(w, jnp.int32)
    y = plsc.bitcast(
        lax.shift_right_logical(bits, jnp.int32(1)) + jnp.int32(0x1FBD1DF5),
        jnp.float32)
    for _ in range(3):
        y = 0.5 * (y + w / y)
    return y


def _bin_stats(N, S1, S2, is_lo):
    d1 = jnp.where(N == 0.0, 1.0, N)
    locb = S1 / d1
    numer = jnp.maximum(S2 - 2.0 * locb * S1 + locb * locb * N, 0.0)
    d2r = N - 1.0
    d2 = jnp.where(d2r == 0.0, 1.0, d2r)
    scaleb = _newton_sqrt(numer / d2 + 1e-5)
    if is_lo:
        lane = lax.iota(jnp.int32, 16)
        locb = jnp.where(lane < 8, 0.0, locb)
        scaleb = jnp.where(lane < 8, 1.0, scaleb)
    return locb, scaleb


def _sc_segment(n_hbm, s1_hbm, s2_hbm, combo_hbm, off_hbm, loc_hbm, scale_hbm,
                n_v, s1_v, s2_v, combo_v, off_v, bins_n, bins_s1, bins_s2,
                loc_tab, scale_tab, oloc_v, oscale_v, sem):
    wid = lax.axis_index("s") * 2 + lax.axis_index("c")

    @pl.when(wid < _B)
    def _():
        row = wid
        cps = [pltpu.async_copy(n_hbm.at[row], n_v, sem),
               pltpu.async_copy(s1_hbm.at[row], s1_v, sem),
               pltpu.async_copy(s2_hbm.at[row], s2_v, sem),
               pltpu.async_copy(combo_hbm.at[row], combo_v, sem),
               pltpu.async_copy(off_hbm.at[row], off_v, sem)]
        for cp in cps:
            cp.wait()

        zero = jnp.zeros((16,), jnp.float32)
        lane = lax.iota(jnp.int32, 16)

        o0 = off_v[pl.ds(0, 16)]
        o1 = off_v[pl.ds(16, 16)]
        o2 = off_v[pl.ds(32, 16)]
        offs = ([o0[i] for i in range(16)] + [o1[i] for i in range(16)]
                + [o2[0]])

        for c in range(32):
            s = offs[c]
            e = offs[c + 1]

            @pl.when(s < e)
            def _(c=c, s=s, e=e):
                k0 = lax.shift_right_logical(s, jnp.int32(4))
                k1 = lax.shift_right_logical(e + jnp.int32(15), jnp.int32(4))

                def body(k, accs):
                    an, a1, a2 = accs
                    sl = pl.ds(k * 16, 16)
                    pos = k * 16 + lane
                    m = jnp.logical_and(pos >= s, pos < e)
                    an = an + jnp.where(m, n_v[sl], zero)
                    a1 = a1 + jnp.where(m, s1_v[sl], zero)
                    a2 = a2 + jnp.where(m, s2_v[sl], zero)
                    return an, a1, a2

                an, a1, a2 = lax.fori_loop(k0, k1, body, (zero, zero, zero))
                slc = pl.ds(c * 16, 16)
                bins_n[slc] = an
                bins_s1[slc] = a1
                bins_s2[slc] = a2

        def row_totals(bins_ref, half):
            tot = zero
            base = lane * 16 + half * 256
            for k in range(16):
                tot = tot + plsc.load_gather(bins_ref, [base + k])
            return tot

        loc_lo, scale_lo = _bin_stats(row_totals(bins_n, 0),
                                      row_totals(bins_s1, 0),
                                      row_totals(bins_s2, 0), True)
        loc_hi, scale_hi = _bin_stats(row_totals(bins_n, 1),
                                      row_totals(bins_s1, 1),
                                      row_totals(bins_s2, 1), False)
        loc_tab[pl.ds(0, 16)] = loc_lo
        loc_tab[pl.ds(16, 16)] = loc_hi
        scale_tab[pl.ds(0, 16)] = scale_lo
        scale_tab[pl.ds(16, 16)] = scale_hi

        def phase2(v, t):
            sl = pl.ds(v * 16, 16)
            c_vec = combo_v[sl]
            oloc_v[sl] = plsc.load_gather(loc_tab, [c_vec])
            oscale_v[sl] = plsc.load_gather(scale_tab, [c_vec])
            return t

        lax.fori_loop(jnp.int32(0), jnp.int32(_NV), phase2, jnp.int32(0))

        co = pltpu.async_copy(oloc_v, loc_hbm.at[row], sem)
        cs = pltpu.async_copy(oscale_v, scale_hbm.at[row], sem)
        co.wait()
        cs.wait()


@jax.jit
def _run(target, observed_mask, sid32, vid32):
    n, s1, s2, combo, off = pl.pallas_call(
        _tc_stats,
        out_shape=(
            jax.ShapeDtypeStruct((_B, _S), jnp.float32),
            jax.ShapeDtypeStruct((_B, _S), jnp.float32),
            jax.ShapeDtypeStruct((_B, _S), jnp.float32),
            jax.ShapeDtypeStruct((_B, _S), jnp.int32),
            jax.ShapeDtypeStruct((_B, 64), jnp.int32),
        ),
    )(target, observed_mask, sid32, vid32)

    mesh = plsc.VectorSubcoreMesh(core_axis_name="c", subcore_axis_name="s")
    seg = pl.kernel(
        _sc_segment,
        mesh=mesh,
        compiler_params=pltpu.CompilerParams(needs_layout_passes=False),
        out_type=(
            jax.ShapeDtypeStruct((_B, _S), jnp.float32),
            jax.ShapeDtypeStruct((_B, _S), jnp.float32),
        ),
        scratch_types=[
            pltpu.VMEM((_S,), jnp.float32),
            pltpu.VMEM((_S,), jnp.float32),
            pltpu.VMEM((_S,), jnp.float32),
            pltpu.VMEM((_S,), jnp.int32),
            pltpu.VMEM((64,), jnp.int32),
            pltpu.VMEM((512,), jnp.float32),
            pltpu.VMEM((512,), jnp.float32),
            pltpu.VMEM((512,), jnp.float32),
            pltpu.VMEM((32,), jnp.float32),
            pltpu.VMEM((32,), jnp.float32),
            pltpu.VMEM((_S,), jnp.float32),
            pltpu.VMEM((_S,), jnp.float32),
            pltpu.SemaphoreType.DMA,
        ],
    )
    loc, scale = seg(n, s1, s2, combo, off)
    return loc[..., None], scale[..., None]


def kernel(target, observed_mask, sample_id, variate_id):
    sid32 = sample_id.astype(jnp.int32)
    vid32 = variate_id.astype(jnp.int32)
    return _run(target, observed_mask.astype(jnp.float32), sid32, vid32)


# speedup vs baseline: 1.1455x; 1.1455x over previous
import functools

import jax
import jax.numpy as jnp
from jax import lax
from jax.experimental import pallas as pl
from jax.experimental.pallas import tpu as pltpu
from jax.experimental.pallas import tpu_sc as plsc

_B, _S, _D = 4, 2048, 64
_NV = _S // 16


def _tc_stats(t_ref, obs_ref, sid_ref, vid_ref, stats_ref, combo_ref,
              off_ref):
    t = t_ref[...]
    obs = obs_ref[...]
    obs_f = jnp.where(obs, jnp.float32(1.0), jnp.float32(0.0))
    tm = t * obs_f
    tt = tm * t
    ones = jnp.full((_B, 1, _D), 1.0, jnp.float32)
    dn = (((2,), (2,)), ((0,), (0,)))
    nT = jax.lax.dot_general(ones, obs_f, dn,
                             preferred_element_type=jnp.float32)
    s1T = jax.lax.dot_general(ones, tm, dn,
                              preferred_element_type=jnp.float32)
    s2T = jax.lax.dot_general(ones, tt, dn,
                              preferred_element_type=jnp.float32)
    stats_ref[...] = jnp.concatenate([nT, s1T, s2T], axis=1)

    combo = sid_ref[...] * 8 + vid_ref[...]
    combo_ref[...] = combo
    iota64 = jax.lax.broadcasted_iota(jnp.int32, (1, 1, 64), 2)
    lt = (combo[:, :, None] < iota64).astype(jnp.float32)
    off_ref[...] = jnp.sum(lt, axis=1).astype(jnp.int32)


def _newton_sqrt(w):
    bits = plsc.bitcast(w, jnp.int32)
    y = plsc.bitcast(
        lax.shift_right_logical(bits, jnp.int32(1)) + jnp.int32(0x1FBD1DF5),
        jnp.float32)
    for _ in range(3):
        y = 0.5 * (y + w / y)
    return y


def _bin_stats(N, S1, S2, is_lo):
    d1 = jnp.where(N == 0.0, 1.0, N)
    locb = S1 / d1
    numer = jnp.maximum(S2 - 2.0 * locb * S1 + locb * locb * N, 0.0)
    d2r = N - 1.0
    d2 = jnp.where(d2r == 0.0, 1.0, d2r)
    scaleb = _newton_sqrt(numer / d2 + 1e-5)
    if is_lo:
        lane = lax.iota(jnp.int32, 16)
        locb = jnp.where(lane < 8, 0.0, locb)
        scaleb = jnp.where(lane < 8, 1.0, scaleb)
    return locb, scaleb


def _sc_segment(stats_hbm, combo_hbm, off_hbm, loc_hbm, scale_hbm,
                stats_v, combo_v, off_v, bins_n, bins_s1, bins_s2,
                loc_tab, scale_tab, oloc_v, oscale_v, sem):
    wid = lax.axis_index("s") * 2 + lax.axis_index("c")

    @pl.when(wid < _B)
    def _():
        row = wid
        cps = [pltpu.async_copy(stats_hbm.at[row], stats_v, sem),
               pltpu.async_copy(combo_hbm.at[row], combo_v, sem),
               pltpu.async_copy(off_hbm.at[row], off_v, sem)]
        for cp in cps:
            cp.wait()

        zero = jnp.zeros((16,), jnp.float32)
        lane = lax.iota(jnp.int32, 16)

        o0 = off_v[pl.ds(0, 16)]
        o1 = off_v[pl.ds(16, 16)]
        o2 = off_v[pl.ds(32, 16)]
        offs = ([o0[i] for i in range(16)] + [o1[i] for i in range(16)]
                + [o2[0]])

        for c in range(32):
            s = offs[c]
            e = offs[c + 1]

            @pl.when(s < e)
            def _(c=c, s=s, e=e):
                k0 = lax.shift_right_logical(s, jnp.int32(4))
                k1 = lax.shift_right_logical(e + jnp.int32(15), jnp.int32(4))

                def body(k, accs):
                    an, a1, a2 = accs
                    sl = pl.ds(k * 16, 16)
                    pos = k * 16 + lane
                    m = jnp.logical_and(pos >= s, pos < e)
                    an = an + jnp.where(m, stats_v[0, sl], zero)
                    a1 = a1 + jnp.where(m, stats_v[1, sl], zero)
                    a2 = a2 + jnp.where(m, stats_v[2, sl], zero)
                    return an, a1, a2

                an, a1, a2 = lax.fori_loop(k0, k1, body, (zero, zero, zero))
                slc = pl.ds(c * 16, 16)
                bins_n[slc] = an
                bins_s1[slc] = a1
                bins_s2[slc] = a2

        def row_totals(bins_ref, half):
            tot = zero
            base = lane * 16 + half * 256
            for k in range(16):
                tot = tot + plsc.load_gather(bins_ref, [base + k])
            return tot

        loc_lo, scale_lo = _bin_stats(row_totals(bins_n, 0),
                                      row_totals(bins_s1, 0),
                                      row_totals(bins_s2, 0), True)
        loc_hi, scale_hi = _bin_stats(row_totals(bins_n, 1),
                                      row_totals(bins_s1, 1),
                                      row_totals(bins_s2, 1), False)
        loc_tab[pl.ds(0, 16)] = loc_lo
        loc_tab[pl.ds(16, 16)] = loc_hi
        scale_tab[pl.ds(0, 16)] = scale_lo
        scale_tab[pl.ds(16, 16)] = scale_hi

        def phase2(v, t):
            sl = pl.ds(v * 16, 16)
            c_vec = combo_v[sl]
            oloc_v[sl] = plsc.load_gather(loc_tab, [c_vec])
            oscale_v[sl] = plsc.load_gather(scale_tab, [c_vec])
            return t

        lax.fori_loop(jnp.int32(0), jnp.int32(_NV), phase2, jnp.int32(0))

        co = pltpu.async_copy(oloc_v, loc_hbm.at[row], sem)
        cs = pltpu.async_copy(oscale_v, scale_hbm.at[row], sem)
        co.wait()
        cs.wait()


@jax.jit
def _run(target, observed_mask, sid32, vid32):
    stats, combo, off = pl.pallas_call(
        _tc_stats,
        out_shape=(
            jax.ShapeDtypeStruct((_B, 3, _S), jnp.float32),
            jax.ShapeDtypeStruct((_B, _S), jnp.int32),
            jax.ShapeDtypeStruct((_B, 64), jnp.int32),
        ),
    )(target, observed_mask, sid32, vid32)

    mesh = plsc.VectorSubcoreMesh(core_axis_name="c", subcore_axis_name="s")
    seg = pl.kernel(
        _sc_segment,
        mesh=mesh,
        compiler_params=pltpu.CompilerParams(needs_layout_passes=False),
        out_type=(
            jax.ShapeDtypeStruct((_B, _S), jnp.float32),
            jax.ShapeDtypeStruct((_B, _S), jnp.float32),
        ),
        scratch_types=[
            pltpu.VMEM((3, _S), jnp.float32),
            pltpu.VMEM((_S,), jnp.int32),
            pltpu.VMEM((64,), jnp.int32),
            pltpu.VMEM((512,), jnp.float32),
            pltpu.VMEM((512,), jnp.float32),
            pltpu.VMEM((512,), jnp.float32),
            pltpu.VMEM((32,), jnp.float32),
            pltpu.VMEM((32,), jnp.float32),
            pltpu.VMEM((_S,), jnp.float32),
            pltpu.VMEM((_S,), jnp.float32),
            pltpu.SemaphoreType.DMA,
        ],
    )
    loc, scale = seg(stats, combo, off)
    return loc[..., None], scale[..., None]


def kernel(target, observed_mask, sample_id, variate_id):
    sid32 = sample_id.astype(jnp.int32)
    vid32 = variate_id.astype(jnp.int32)
    return _run(target, observed_mask, sid32, vid32)


# speedup vs baseline: 1.1500x; 1.0040x over previous
import functools

import jax
import jax.numpy as jnp
from jax import lax
from jax.experimental import pallas as pl
from jax.experimental.pallas import tpu as pltpu
from jax.experimental.pallas import tpu_sc as plsc

_B, _S, _D = 4, 2048, 64
_NV = _S // 16


def _tc_stats(t_ref, obs_ref, sid_ref, vid_ref, stats_ref, combo_ref,
              off_ref):
    t = t_ref[...]
    obs = obs_ref[...]
    obs_f = jnp.where(obs, jnp.float32(1.0), jnp.float32(0.0))
    tm = t * obs_f
    tt = tm * t
    ones = jnp.full((_B, 1, _D), 1.0, jnp.float32)
    dn = (((2,), (2,)), ((0,), (0,)))
    nT = jax.lax.dot_general(ones, obs_f, dn,
                             preferred_element_type=jnp.float32)
    s1T = jax.lax.dot_general(ones, tm, dn,
                              preferred_element_type=jnp.float32)
    s2T = jax.lax.dot_general(ones, tt, dn,
                              preferred_element_type=jnp.float32)
    stats_ref[...] = jnp.concatenate([nT, s1T, s2T], axis=1)

    combo = sid_ref[...] * 8 + vid_ref[...]
    combo_ref[...] = combo
    iota64 = jax.lax.broadcasted_iota(jnp.int32, (1, 1, 64), 2)
    lt = (combo[:, :, None] < iota64).astype(jnp.float32)
    off_ref[...] = jnp.sum(lt, axis=1).astype(jnp.int32)


def _newton_sqrt(w):
    bits = plsc.bitcast(w, jnp.int32)
    y = plsc.bitcast(
        lax.shift_right_logical(bits, jnp.int32(1)) + jnp.int32(0x1FBD1DF5),
        jnp.float32)
    for _ in range(3):
        y = 0.5 * (y + w / y)
    return y


def _bin_stats(N, S1, S2, is_lo):
    d1 = jnp.where(N == 0.0, 1.0, N)
    locb = S1 / d1
    numer = jnp.maximum(S2 - 2.0 * locb * S1 + locb * locb * N, 0.0)
    d2r = N - 1.0
    d2 = jnp.where(d2r == 0.0, 1.0, d2r)
    scaleb = _newton_sqrt(numer / d2 + 1e-5)
    if is_lo:
        lane = lax.iota(jnp.int32, 16)
        locb = jnp.where(lane < 8, 0.0, locb)
        scaleb = jnp.where(lane < 8, 1.0, scaleb)
    return locb, scaleb


def _sc_segment(stats_hbm, combo_hbm, off_hbm, loc_hbm, scale_hbm,
                stats_v, combo_v, off_v, bins_n, bins_s1, bins_s2,
                tot_v, tmp8, tab_v, oloc_v, oscale_v, shared, sem):
    cid = lax.axis_index("c")
    sid = lax.axis_index("s")
    row = cid * 2 + lax.shift_right_logical(sid, jnp.int32(3))
    sub = jnp.bitwise_and(sid, jnp.int32(7))
    c0 = sub * 256

    cps = [pltpu.async_copy(stats_hbm.at[row], stats_v, sem),
           pltpu.async_copy(combo_hbm.at[row], combo_v, sem),
           pltpu.async_copy(off_hbm.at[row], off_v, sem)]

    zero = jnp.zeros((16,), jnp.float32)
    lane = lax.iota(jnp.int32, 16)

    for i in range(32):
        sl0 = pl.ds(i * 16, 16)
        bins_n[sl0] = zero
        bins_s1[sl0] = zero
        bins_s2[sl0] = zero

    for cp in cps:
        cp.wait()

    o0 = off_v[pl.ds(0, 16)]
    o1 = off_v[pl.ds(16, 16)]
    o2 = off_v[pl.ds(32, 16)]
    offs = ([o0[i] for i in range(16)] + [o1[i] for i in range(16)]
            + [o2[0]])

    for c in range(32):
        cs = jnp.maximum(offs[c], c0)
        ce = jnp.minimum(offs[c + 1], c0 + 256)

        @pl.when(cs < ce)
        def _(c=c, cs=cs, ce=ce):
            k0 = lax.shift_right_logical(cs, jnp.int32(4))
            k1 = lax.shift_right_logical(ce + jnp.int32(15), jnp.int32(4))

            def body(k, accs):
                an, a1, a2 = accs
                sl = pl.ds(k * 16, 16)
                pos = k * 16 + lane
                m = jnp.logical_and(pos >= cs, pos < ce)
                an = an + jnp.where(m, stats_v[0, sl], zero)
                a1 = a1 + jnp.where(m, stats_v[1, sl], zero)
                a2 = a2 + jnp.where(m, stats_v[2, sl], zero)
                return an, a1, a2

            an, a1, a2 = lax.fori_loop(k0, k1, body, (zero, zero, zero))
            slc = pl.ds(c * 16, 16)
            bins_n[slc] = an
            bins_s1[slc] = a1
            bins_s2[slc] = a2

    def row_totals(bins_ref, half):
        tot = zero
        base = lane * 16 + half * 256
        for k in range(16):
            tot = tot + plsc.load_gather(bins_ref, [base + k])
        return tot

    tot_v[pl.ds(0, 16)] = row_totals(bins_n, 0)
    tot_v[pl.ds(16, 16)] = row_totals(bins_n, 1)
    tot_v[pl.ds(32, 16)] = row_totals(bins_s1, 0)
    tot_v[pl.ds(48, 16)] = row_totals(bins_s1, 1)
    tot_v[pl.ds(64, 16)] = row_totals(bins_s2, 0)
    tot_v[pl.ds(80, 16)] = row_totals(bins_s2, 1)
    wid = cid * 16 + sid
    pltpu.sync_copy(tot_v, shared.at[pl.ds(wid * 96, 96)])
    plsc.subcore_barrier()

    half8 = cid * 16 + lax.shift_right_logical(sid, jnp.int32(3)) * 8
    for h in (0, 8, 16, 24):
        @pl.when(half8 == h)
        def _(h=h):
            pltpu.sync_copy(shared.at[pl.ds(h * 96, 8 * 96)], tmp8)

    def combined(col):
        tot = zero
        for j in range(8):
            tot = tot + tmp8[pl.ds(j * 96 + col * 16, 16)]
        return tot

    loc_lo, scale_lo = _bin_stats(combined(0), combined(2), combined(4),
                                  True)
    loc_hi, scale_hi = _bin_stats(combined(1), combined(3), combined(5),
                                  False)
    tab_v[pl.ds(0, 16)] = loc_lo
    tab_v[pl.ds(16, 16)] = loc_hi
    tab_v[pl.ds(32, 16)] = scale_lo
    tab_v[pl.ds(48, 16)] = scale_hi

    def phase2(v, t):
        c_vec = combo_v[pl.ds(c0 + v * 16, 16)]
        sl = pl.ds(v * 16, 16)
        oloc_v[sl] = plsc.load_gather(tab_v, [c_vec])
        oscale_v[sl] = plsc.load_gather(tab_v, [c_vec + 32])
        return t

    lax.fori_loop(jnp.int32(0), jnp.int32(16), phase2, jnp.int32(0))

    chunk = row * 8 + sub
    co = pltpu.async_copy(oloc_v, loc_hbm.at[chunk], sem)
    cs2 = pltpu.async_copy(oscale_v, scale_hbm.at[chunk], sem)
    co.wait()
    cs2.wait()


@jax.jit
def _run(target, observed_mask, sid32, vid32):
    stats, combo, off = pl.pallas_call(
        _tc_stats,
        out_shape=(
            jax.ShapeDtypeStruct((_B, 3, _S), jnp.float32),
            jax.ShapeDtypeStruct((_B, _S), jnp.int32),
            jax.ShapeDtypeStruct((_B, 64), jnp.int32),
        ),
    )(target, observed_mask, sid32, vid32)

    mesh = plsc.VectorSubcoreMesh(core_axis_name="c", subcore_axis_name="s")
    seg = pl.kernel(
        _sc_segment,
        mesh=mesh,
        compiler_params=pltpu.CompilerParams(needs_layout_passes=False),
        out_type=(
            jax.ShapeDtypeStruct((32, 256), jnp.float32),
            jax.ShapeDtypeStruct((32, 256), jnp.float32),
        ),
        scratch_types=[
            pltpu.VMEM((3, _S), jnp.float32),
            pltpu.VMEM((_S,), jnp.int32),
            pltpu.VMEM((64,), jnp.int32),
            pltpu.VMEM((512,), jnp.float32),
            pltpu.VMEM((512,), jnp.float32),
            pltpu.VMEM((512,), jnp.float32),
            pltpu.VMEM((96,), jnp.float32),
            pltpu.VMEM((768,), jnp.float32),
            pltpu.VMEM((64,), jnp.float32),
            pltpu.VMEM((256,), jnp.float32),
            pltpu.VMEM((256,), jnp.float32),
            pltpu.VMEM_SHARED((3072,), jnp.float32),
            pltpu.SemaphoreType.DMA,
        ],
    )
    loc, scale = seg(stats, combo, off)
    loc = loc.reshape(_B, _S, 1)
    scale = scale.reshape(_B, _S, 1)
    return loc, scale


def kernel(target, observed_mask, sample_id, variate_id):
    sid32 = sample_id.astype(jnp.int32)
    vid32 = variate_id.astype(jnp.int32)
    return _run(target, observed_mask, sid32, vid32)


# speedup vs baseline: 1.1726x; 1.0196x over previous
import jax
import jax.numpy as jnp
from jax import lax
from jax.experimental import pallas as pl
from jax.experimental.pallas import tpu as pltpu
from jax.experimental.pallas import tpu_sc as plsc

_B, _S, _D = 4, 2048, 64
_NV = _S // 16


def _tc_stats(t_ref, obs_ref, sid_ref, vid_ref, stats_ref, combo_ref,
              off_ref):
    t = t_ref[...]
    obs = obs_ref[...]
    obs_f = jnp.where(obs, jnp.float32(1.0), jnp.float32(0.0))
    tm = t * obs_f
    tt = tm * t
    ones = jnp.full((_B, 1, _D), 1.0, jnp.float32)
    dn = (((2,), (2,)), ((0,), (0,)))
    nT = jax.lax.dot_general(ones, obs_f, dn,
                             preferred_element_type=jnp.float32)
    s1T = jax.lax.dot_general(ones, tm, dn,
                              preferred_element_type=jnp.float32)
    s2T = jax.lax.dot_general(ones, tt, dn,
                              preferred_element_type=jnp.float32)
    stats_ref[...] = jnp.concatenate([nT, s1T, s2T], axis=1)

    combo = sid_ref[...] * 8 + vid_ref[...]
    combo_ref[...] = combo
    iota64 = jax.lax.broadcasted_iota(jnp.int32, (1, 1, 64), 2)
    lt = (combo[:, :, None] < iota64).astype(jnp.float32)
    off_ref[...] = jnp.sum(lt, axis=1).astype(jnp.int32)


def _newton_sqrt(w):
    bits = plsc.bitcast(w, jnp.int32)
    y = plsc.bitcast(
        lax.shift_right_logical(bits, jnp.int32(1)) + jnp.int32(0x1FBD1DF5),
        jnp.float32)
    for _ in range(3):
        y = 0.5 * (y + w / y)
    return y


def _bin_stats(N, S1, S2, is_lo):
    d1 = jnp.where(N == 0.0, 1.0, N)
    locb = S1 / d1
    numer = jnp.maximum(S2 - 2.0 * locb * S1 + locb * locb * N, 0.0)
    d2r = N - 1.0
    d2 = jnp.where(d2r == 0.0, 1.0, d2r)
    scaleb = _newton_sqrt(numer / d2 + 1e-5)
    if is_lo:
        lane = lax.iota(jnp.int32, 16)
        locb = jnp.where(lane < 8, 0.0, locb)
        scaleb = jnp.where(lane < 8, 1.0, scaleb)
    return locb, scaleb


def _sc_segment(stats_hbm, combo_hbm, off_hbm, loc_hbm, scale_hbm,
                stats_v, combo_v, off_v, bins_n, bins_s1, bins_s2,
                tot_v, tmp8, tab_v, oloc_v, oscale_v, shared, sem):
    cid = lax.axis_index("c")
    sid = lax.axis_index("s")
    row = cid * 2 + lax.shift_right_logical(sid, jnp.int32(3))
    sub = jnp.bitwise_and(sid, jnp.int32(7))
    c0 = sub * 256

    cps = [pltpu.async_copy(stats_hbm.at[row, :, pl.ds(c0, 256)], stats_v,
                            sem),
           pltpu.async_copy(combo_hbm.at[row, pl.ds(c0, 256)], combo_v, sem),
           pltpu.async_copy(off_hbm.at[row], off_v, sem)]

    zero = jnp.zeros((16,), jnp.float32)
    lane = lax.iota(jnp.int32, 16)

    for i in range(32):
        sl0 = pl.ds(i * 16, 16)
        bins_n[sl0] = zero
        bins_s1[sl0] = zero
        bins_s2[sl0] = zero

    for cp in cps:
        cp.wait()

    o0 = off_v[pl.ds(0, 16)]
    o1 = off_v[pl.ds(16, 16)]
    o2 = off_v[pl.ds(32, 16)]
    offs = ([o0[i] for i in range(16)] + [o1[i] for i in range(16)]
            + [o2[0]])

    for c in range(32):
        cs = jnp.maximum(offs[c], c0)
        ce = jnp.minimum(offs[c + 1], c0 + 256)

        @pl.when(cs < ce)
        def _(c=c, cs=cs, ce=ce):
            k0 = lax.shift_right_logical(cs - c0, jnp.int32(4))
            k1 = lax.shift_right_logical(ce - c0 + jnp.int32(15), jnp.int32(4))

            def body(k, accs):
                an, a1, a2 = accs
                sl = pl.ds(k * 16, 16)
                pos = c0 + k * 16 + lane
                m = jnp.logical_and(pos >= cs, pos < ce)
                an = an + jnp.where(m, stats_v[0, sl], zero)
                a1 = a1 + jnp.where(m, stats_v[1, sl], zero)
                a2 = a2 + jnp.where(m, stats_v[2, sl], zero)
                return an, a1, a2

            an, a1, a2 = lax.fori_loop(k0, k1, body, (zero, zero, zero))
            slc = pl.ds(c * 16, 16)
            bins_n[slc] = an
            bins_s1[slc] = a1
            bins_s2[slc] = a2

    def row_totals(bins_ref, half):
        tot = zero
        base = lane * 16 + half * 256
        for k in range(16):
            tot = tot + plsc.load_gather(bins_ref, [base + k])
        return tot

    tot_v[pl.ds(0, 16)] = row_totals(bins_n, 0)
    tot_v[pl.ds(16, 16)] = row_totals(bins_n, 1)
    tot_v[pl.ds(32, 16)] = row_totals(bins_s1, 0)
    tot_v[pl.ds(48, 16)] = row_totals(bins_s1, 1)
    tot_v[pl.ds(64, 16)] = row_totals(bins_s2, 0)
    tot_v[pl.ds(80, 16)] = row_totals(bins_s2, 1)
    wid = cid * 16 + sid
    pltpu.sync_copy(tot_v, shared.at[pl.ds(wid * 96, 96)])
    plsc.subcore_barrier()

    half8 = cid * 16 + lax.shift_right_logical(sid, jnp.int32(3)) * 8
    for h in (0, 8, 16, 24):
        @pl.when(half8 == h)
        def _(h=h):
            pltpu.sync_copy(shared.at[pl.ds(h * 96, 8 * 96)], tmp8)

    def combined(col):
        tot = zero
        for j in range(8):
            tot = tot + tmp8[pl.ds(j * 96 + col * 16, 16)]
        return tot

    loc_lo, scale_lo = _bin_stats(combined(0), combined(2), combined(4),
                                  True)
    loc_hi, scale_hi = _bin_stats(combined(1), combined(3), combined(5),
                                  False)
    tab_v[pl.ds(0, 16)] = loc_lo
    tab_v[pl.ds(16, 16)] = loc_hi
    tab_v[pl.ds(32, 16)] = scale_lo
    tab_v[pl.ds(48, 16)] = scale_hi

    def phase2(v, t):
        c_vec = combo_v[pl.ds(v * 16, 16)]
        sl = pl.ds(v * 16, 16)
        oloc_v[sl] = plsc.load_gather(tab_v, [c_vec])
        oscale_v[sl] = plsc.load_gather(tab_v, [c_vec + 32])
        return t

    lax.fori_loop(jnp.int32(0), jnp.int32(16), phase2, jnp.int32(0))

    chunk = row * 8 + sub
    co = pltpu.async_copy(oloc_v, loc_hbm.at[chunk], sem)
    cs2 = pltpu.async_copy(oscale_v, scale_hbm.at[chunk], sem)
    co.wait()
    cs2.wait()


@jax.jit
def _run(target, observed_mask, sid32, vid32):
    stats, combo, off = pl.pallas_call(
        _tc_stats,
        out_shape=(
            jax.ShapeDtypeStruct((_B, 3, _S), jnp.float32),
            jax.ShapeDtypeStruct((_B, _S), jnp.int32),
            jax.ShapeDtypeStruct((_B, 64), jnp.int32),
        ),
    )(target, observed_mask, sid32, vid32)

    mesh = plsc.VectorSubcoreMesh(core_axis_name="c", subcore_axis_name="s")
    seg = pl.kernel(
        _sc_segment,
        mesh=mesh,
        compiler_params=pltpu.CompilerParams(needs_layout_passes=False),
        out_type=(
            jax.ShapeDtypeStruct((32, 256), jnp.float32),
            jax.ShapeDtypeStruct((32, 256), jnp.float32),
        ),
        scratch_types=[
            pltpu.VMEM((3, 256), jnp.float32),
            pltpu.VMEM((256,), jnp.int32),
            pltpu.VMEM((64,), jnp.int32),
            pltpu.VMEM((512,), jnp.float32),
            pltpu.VMEM((512,), jnp.float32),
            pltpu.VMEM((512,), jnp.float32),
            pltpu.VMEM((96,), jnp.float32),
            pltpu.VMEM((768,), jnp.float32),
            pltpu.VMEM((64,), jnp.float32),
            pltpu.VMEM((256,), jnp.float32),
            pltpu.VMEM((256,), jnp.float32),
            pltpu.VMEM_SHARED((3072,), jnp.float32),
            pltpu.SemaphoreType.DMA,
        ],
    )
    loc, scale = seg(stats, combo, off)
    loc = loc.reshape(_B, _S, 1)
    scale = scale.reshape(_B, _S, 1)
    return loc, scale


def kernel(target, observed_mask, sample_id, variate_id):
    sid32 = sample_id.astype(jnp.int32)
    vid32 = variate_id.astype(jnp.int32)
    return _run(target, observed_mask, sid32, vid32)


# speedup vs baseline: 1.1745x; 1.0016x over previous
import jax
import jax.numpy as jnp
from jax import lax
from jax.experimental import pallas as pl
from jax.experimental.pallas import tpu as pltpu
from jax.experimental.pallas import tpu_sc as plsc

_B, _S, _D = 4, 2048, 64


def _tc_stats(t_ref, obs_ref, sid_ref, vid_ref, stats_ref, combo_ref,
              off_ref):
    t = t_ref[...]
    obs = obs_ref[...]
    obs_f = jnp.where(obs, jnp.float32(1.0), jnp.float32(0.0))
    tm = t * obs_f
    tt = tm * t
    ones = jnp.full((_B, 1, _D), 1.0, jnp.float32)
    dn = (((2,), (2,)), ((0,), (0,)))
    nT = jax.lax.dot_general(ones, obs_f, dn,
                             preferred_element_type=jnp.float32)
    s1T = jax.lax.dot_general(ones, tm, dn,
                              preferred_element_type=jnp.float32)
    s2T = jax.lax.dot_general(ones, tt, dn,
                              preferred_element_type=jnp.float32)
    stats_ref[...] = jnp.concatenate([nT, s1T, s2T], axis=1)

    combo = sid_ref[...] * 8 + vid_ref[...]
    combo_ref[...] = combo
    iota64 = jax.lax.broadcasted_iota(jnp.int32, (1, 1, 64), 2)
    lt = (combo[:, :, None] < iota64).astype(jnp.float32)
    off_ref[...] = jnp.sum(lt, axis=1).astype(jnp.int32)


def _newton_sqrt(w):
    bits = plsc.bitcast(w, jnp.int32)
    y = plsc.bitcast(
        lax.shift_right_logical(bits, jnp.int32(1)) + jnp.int32(0x1FBD1DF5),
        jnp.float32)
    for _ in range(3):
        y = 0.5 * (y + w / y)
    return y


def _bin_stats(N, S1, S2, is_lo):
    d1 = jnp.where(N == 0.0, 1.0, N)
    locb = S1 / d1
    numer = jnp.maximum(S2 - 2.0 * locb * S1 + locb * locb * N, 0.0)
    d2r = N - 1.0
    d2 = jnp.where(d2r == 0.0, 1.0, d2r)
    scaleb = _newton_sqrt(numer / d2 + 1e-5)
    if is_lo:
        lane = lax.iota(jnp.int32, 16)
        locb = jnp.where(lane < 8, 0.0, locb)
        scaleb = jnp.where(lane < 8, 1.0, scaleb)
    return locb, scaleb


def _sc_segment(stats_hbm, combo_hbm, off_hbm, loc_hbm, scale_hbm,
                stats_v, combo_v, off_v, bins_n, bins_s1, bins_s2,
                tot_v, tmp8, tab_v, oloc_v, oscale_v, shared, sem):
    cid = lax.axis_index("c")
    sid = lax.axis_index("s")
    row = cid * 2 + lax.shift_right_logical(sid, jnp.int32(3))
    sub = jnp.bitwise_and(sid, jnp.int32(7))
    c0 = sub * 256

    cps = [pltpu.async_copy(stats_hbm.at[row, :, pl.ds(c0, 256)], stats_v,
                            sem),
           pltpu.async_copy(combo_hbm.at[row, pl.ds(c0, 256)], combo_v, sem),
           pltpu.async_copy(off_hbm.at[row], off_v, sem)]

    zero = jnp.zeros((16,), jnp.float32)
    lane = lax.iota(jnp.int32, 16)

    for i in range(32):
        sl0 = pl.ds(i * 16, 16)
        bins_n[sl0] = zero
        bins_s1[sl0] = zero
        bins_s2[sl0] = zero

    for cp in cps:
        cp.wait()

    o0 = off_v[pl.ds(0, 16)]
    o1 = off_v[pl.ds(16, 16)]
    o2 = off_v[pl.ds(32, 16)]
    offs = ([o0[i] for i in range(16)] + [o1[i] for i in range(16)]
            + [o2[0]])

    for c in range(32):
        cs = jnp.maximum(offs[c], c0)
        ce = jnp.minimum(offs[c + 1], c0 + 256)

        @pl.when(cs < ce)
        def _(c=c, cs=cs, ce=ce):
            k0 = lax.shift_right_logical(cs - c0, jnp.int32(4))
            k1 = lax.shift_right_logical(ce - c0 + jnp.int32(15), jnp.int32(4))

            def body(k, accs):
                an, a1, a2 = accs
                sl = pl.ds(k * 16, 16)
                pos = c0 + k * 16 + lane
                m = jnp.logical_and(pos >= cs, pos < ce)
                an = an + jnp.where(m, stats_v[0, sl], zero)
                a1 = a1 + jnp.where(m, stats_v[1, sl], zero)
                a2 = a2 + jnp.where(m, stats_v[2, sl], zero)
                return an, a1, a2

            an, a1, a2 = lax.fori_loop(k0, k1, body, (zero, zero, zero))
            slc = pl.ds(c * 16, 16)
            bins_n[slc] = an
            bins_s1[slc] = a1
            bins_s2[slc] = a2

    def row_totals(bins_ref, half):
        tot = zero
        base = lane * 16 + half * 256
        for k in range(16):
            tot = tot + plsc.load_gather(bins_ref, [base + k])
        return tot

    tot_v[pl.ds(0, 16)] = row_totals(bins_n, 0)
    tot_v[pl.ds(16, 16)] = row_totals(bins_n, 1)
    tot_v[pl.ds(32, 16)] = row_totals(bins_s1, 0)
    tot_v[pl.ds(48, 16)] = row_totals(bins_s1, 1)
    tot_v[pl.ds(64, 16)] = row_totals(bins_s2, 0)
    tot_v[pl.ds(80, 16)] = row_totals(bins_s2, 1)
    wid = cid * 16 + sid
    pltpu.sync_copy(tot_v, shared.at[pl.ds(wid * 96, 96)])
    plsc.subcore_barrier()

    half8 = cid * 16 + lax.shift_right_logical(sid, jnp.int32(3)) * 8
    for h in (0, 8, 16, 24):
        @pl.when(half8 == h)
        def _(h=h):
            pltpu.sync_copy(shared.at[pl.ds(h * 96, 8 * 96)], tmp8)

    def combined(col):
        tot = zero
        for j in range(8):
            tot = tot + tmp8[pl.ds(j * 96 + col * 16, 16)]
        return tot

    loc_lo, scale_lo = _bin_stats(combined(0), combined(2), combined(4),
                                  True)
    loc_hi, scale_hi = _bin_stats(combined(1), combined(3), combined(5),
                                  False)
    tab_v[pl.ds(0, 16)] = loc_lo
    tab_v[pl.ds(16, 16)] = loc_hi
    tab_v[pl.ds(32, 16)] = scale_lo
    tab_v[pl.ds(48, 16)] = scale_hi

    def phase2(v, t):
        c_vec = combo_v[pl.ds(v * 16, 16)]
        sl = pl.ds(v * 16, 16)
        oloc_v[sl] = plsc.load_gather(tab_v, [c_vec])
        oscale_v[sl] = plsc.load_gather(tab_v, [c_vec + 32])
        return t

    lax.fori_loop(jnp.int32(0), jnp.int32(16), phase2, jnp.int32(0))

    chunk = row * 8 + sub
    co = pltpu.async_copy(oloc_v, loc_hbm.at[chunk], sem)
    cs2 = pltpu.async_copy(oscale_v, scale_hbm.at[chunk], sem)
    co.wait()
    cs2.wait()


@jax.jit
def _run(target, observed_mask, sid32, vid32):
    stats, combo, off = pl.pallas_call(
        _tc_stats,
        out_shape=(
            jax.ShapeDtypeStruct((_B, 3, _S), jnp.float32),
            jax.ShapeDtypeStruct((_B, _S), jnp.int32),
            jax.ShapeDtypeStruct((_B, 64), jnp.int32),
        ),
    )(target, observed_mask, sid32, vid32)

    mesh = plsc.VectorSubcoreMesh(core_axis_name="c", subcore_axis_name="s")
    seg = pl.kernel(
        _sc_segment,
        mesh=mesh,
        compiler_params=pltpu.CompilerParams(needs_layout_passes=False),
        out_type=(
            jax.ShapeDtypeStruct((32, 256), jnp.float32),
            jax.ShapeDtypeStruct((32, 256), jnp.float32),
        ),
        scratch_types=[
            pltpu.VMEM((3, 256), jnp.float32),
            pltpu.VMEM((256,), jnp.int32),
            pltpu.VMEM((64,), jnp.int32),
            pltpu.VMEM((512,), jnp.float32),
            pltpu.VMEM((512,), jnp.float32),
            pltpu.VMEM((512,), jnp.float32),
            pltpu.VMEM((96,), jnp.float32),
            pltpu.VMEM((768,), jnp.float32),
            pltpu.VMEM((64,), jnp.float32),
            pltpu.VMEM((256,), jnp.float32),
            pltpu.VMEM((256,), jnp.float32),
            pltpu.VMEM_SHARED((3072,), jnp.float32),
            pltpu.SemaphoreType.DMA,
        ],
    )
    loc, scale = seg(stats, combo, off)
    loc = loc.reshape(_B, _S, 1)
    scale = scale.reshape(_B, _S, 1)
    return loc, scale


def kernel(target, observed_mask, sample_id, variate_id):
    sid32 = sample_id.astype(jnp.int32)
    vid32 = variate_id.astype(jnp.int32)
    return _run(target, observed_mask, sid32, vid32)
